# baseline (speedup 1.0000x reference)
import functools

import jax
import jax.numpy as jnp
from jax import lax
from jax.experimental import pallas as pl
from jax.experimental.pallas import tpu as pltpu
from jax.experimental.pallas import tpu_sc as plsc

V = 1000000
EMBED = 32
B = 4096

_info = plsc.get_sparse_core_info()
_NC, _NS = _info.num_cores, _info.num_subcores
_NW = _NC * _NS
_B_PER_W = B // _NW


def _sc_gather(table, center_id, context_id):
    mesh = plsc.VectorSubcoreMesh(core_axis_name="c", subcore_axis_name="s")

    @functools.partial(
        pl.kernel,
        mesh=mesh,
        out_type=[
            jax.ShapeDtypeStruct((B, EMBED), jnp.float32),
            jax.ShapeDtypeStruct((B, EMBED), jnp.float32),
        ],
        scratch_types=[
            pltpu.VMEM((_B_PER_W,), jnp.int32),
            pltpu.VMEM((_B_PER_W,), jnp.int32),
            pltpu.VMEM((_B_PER_W, EMBED), jnp.float32),
            pltpu.VMEM((_B_PER_W, EMBED), jnp.float32),
            pltpu.SemaphoreType.DMA,
        ],
        compiler_params=pltpu.CompilerParams(use_tc_tiling_on_sc=False),
    )
    def gather_kernel(table_hbm, cen_hbm, ctx_hbm, cen_out, ctx_out,
                      cen_idx_v, ctx_idx_v, cen_rows_v, ctx_rows_v, sem):
        wid = lax.axis_index("s") * _NC + lax.axis_index("c")
        base = wid * _B_PER_W
        pltpu.sync_copy(cen_hbm.at[pl.ds(base, _B_PER_W)], cen_idx_v)
        pltpu.sync_copy(ctx_hbm.at[pl.ds(base, _B_PER_W)], ctx_idx_v)
        cp1 = pltpu.async_copy(table_hbm.at[cen_idx_v], cen_rows_v, sem)
        cp2 = pltpu.async_copy(table_hbm.at[ctx_idx_v], ctx_rows_v, sem)
        cp1.wait()
        cp2.wait()
        pltpu.sync_copy(cen_rows_v, cen_out.at[pl.ds(base, _B_PER_W)])
        pltpu.sync_copy(ctx_rows_v, ctx_out.at[pl.ds(base, _B_PER_W)])

    return gather_kernel(table, center_id, context_id)


def _log_sigmoid(x):
    return jnp.minimum(x, 0.0) - jnp.log1p(jnp.exp(-jnp.abs(x)))


def _mm_body(cen_ref, ctx_ref, out_ref):
    prod = lax.dot_general(
        cen_ref[...], ctx_ref[...],
        (((1,), (1,)), ((), ())),
        preferred_element_type=jnp.float32,
    )
    out_ref[...] = _log_sigmoid(prod)


_BM = 512


def _tc_matmul(center_emb, context_emb):
    return pl.pallas_call(
        _mm_body,
        grid=(B // _BM,),
        in_specs=[
            pl.BlockSpec((_BM, EMBED), lambda i: (i, 0)),
            pl.BlockSpec((B, EMBED), lambda i: (0, 0)),
        ],
        out_specs=pl.BlockSpec((_BM, B), lambda i: (i, 0)),
        out_shape=jax.ShapeDtypeStruct((B, B), jnp.float32),
    )(center_emb, context_emb)


@jax.jit
def kernel(center_id, context_id, embeddings):
    center_emb, context_emb = _sc_gather(
        embeddings, center_id.astype(jnp.int32), context_id.astype(jnp.int32))
    return _tc_matmul(center_emb, context_emb)


# speedup vs baseline: 1.6693x; 1.6693x over previous
import jax
import jax.numpy as jnp
from jax import lax
from jax.experimental import pallas as pl
from jax.experimental.pallas import tpu as pltpu

V = 1000000
EMBED = 32
B = 4096
NIDX = 2 * B

_GBLK = 128
_GSTEPS = NIDX // _GBLK
_CHUNK = 8


def _gather_body(cb_smem, rr_ref, emb_t_hbm, out_ref, buf, sem):
    s = pl.program_id(0)
    nsteps = pl.num_programs(0)

    def fire(step, slot):
        for k in range(_GBLK):
            pltpu.make_async_copy(
                emb_t_hbm.at[
                    :, pl.ds(pl.multiple_of(cb_smem[step * _GBLK + k], 128),
                             128)],
                buf.at[slot, k],
                sem.at[slot],
            ).start()

    @pl.when(s == 0)
    def _():
        fire(s, 0)

    @pl.when(s + 1 < nsteps)
    def _():
        fire(s + 1, (s + 1) % 2)

    slot = s % 2
    for k in range(_GBLK):
        pltpu.make_async_copy(
            emb_t_hbm.at[:, pl.ds(0, 128)],
            buf.at[slot, k],
            sem.at[slot],
        ).wait()
    for k in range(0, _GBLK, _CHUNK):
        chunk = buf[slot, pl.ds(k, _CHUNK)]
        idxc = rr_ref[0, pl.ds(k, _CHUNK), :][..., None]
        rows = jnp.take_along_axis(chunk, idxc, axis=2)[..., 0]
        out_ref[pl.ds(k, _CHUNK), :] = rows


def _tc_gather(emb_t, cb, rr_b):
    grid_spec = pltpu.PrefetchScalarGridSpec(
        num_scalar_prefetch=1,
        grid=(_GSTEPS,),
        in_specs=[
            pl.BlockSpec((1, _GBLK, EMBED), lambda s, cb: (s, 0, 0)),
            pl.BlockSpec(memory_space=pl.ANY),
        ],
        out_specs=pl.BlockSpec((_GBLK, EMBED), lambda s, cb: (s, 0)),
        scratch_shapes=[
            pltpu.VMEM((2, _GBLK, EMBED, 128), jnp.float32),
            pltpu.SemaphoreType.DMA((2,)),
        ],
    )
    return pl.pallas_call(
        _gather_body,
        grid_spec=grid_spec,
        out_shape=jax.ShapeDtypeStruct((NIDX, EMBED), jnp.float32),
    )(cb, rr_b, emb_t)


def _log_sigmoid(x):
    return jnp.minimum(x, 0.0) - jnp.log1p(jnp.exp(-jnp.abs(x)))


def _mm_body(cen_ref, ctx_ref, out_ref):
    prod = lax.dot_general(
        cen_ref[...], ctx_ref[...],
        (((1,), (1,)), ((), ())),
        preferred_element_type=jnp.float32,
    )
    out_ref[...] = _log_sigmoid(prod)


_BM = 512


def _tc_matmul(cen, ctx):
    return pl.pallas_call(
        _mm_body,
        grid=(B // _BM,),
        in_specs=[
            pl.BlockSpec((_BM, EMBED), lambda i: (i, 0)),
            pl.BlockSpec((B, EMBED), lambda i: (0, 0)),
        ],
        out_specs=pl.BlockSpec((_BM, B), lambda i: (i, 0)),
        out_shape=jax.ShapeDtypeStruct((B, B), jnp.float32),
    )(cen, ctx)


@jax.jit
def kernel(center_id, context_id, embeddings):
    ids = jnp.concatenate([center_id.astype(jnp.int32),
                           context_id.astype(jnp.int32)])
    cb = (ids // 128) * 128
    rr_b = jnp.broadcast_to(
        (ids % 128).reshape(_GSTEPS, _GBLK)[:, :, None],
        (_GSTEPS, _GBLK, EMBED))
    out = _tc_gather(embeddings.T, cb, rr_b)
    return _tc_matmul(out[:B], out[B:])


# speedup vs baseline: 2.2830x; 1.3676x over previous
import jax
import jax.numpy as jnp
from jax import lax
from jax.experimental import pallas as pl
from jax.experimental.pallas import tpu as pltpu

V = 1000000
EMBED = 32
B = 4096
NIDX = 2 * B

_GBLK = 128
_GSTEPS = NIDX // _GBLK
_CHUNK = 64


def _gather_body(cb_smem, rr_ref, emb_t_hbm, out_ref, buf, sem):
    s = pl.program_id(0)
    nsteps = pl.num_programs(0)

    def fire(step, slot):
        for k in range(_GBLK):
            pltpu.make_async_copy(
                emb_t_hbm.at[
                    :, pl.ds(pl.multiple_of(cb_smem[step * _GBLK + k], 128),
                             128)],
                buf.at[slot, k],
                sem.at[slot],
            ).start()

    @pl.when(s == 0)
    def _():
        fire(s, 0)

    @pl.when(s + 1 < nsteps)
    def _():
        fire(s + 1, (s + 1) % 2)

    slot = s % 2
    for k in range(_GBLK):
        pltpu.make_async_copy(
            emb_t_hbm.at[:, pl.ds(0, 128)],
            buf.at[slot, k],
            sem.at[slot],
        ).wait()
    for k in range(0, _GBLK, _CHUNK):
        chunk = buf[slot, pl.ds(k, _CHUNK)]
        idxc = rr_ref[0, pl.ds(k, _CHUNK), :][..., None]
        rows = jnp.take_along_axis(chunk, idxc, axis=2)[..., 0]
        out_ref[pl.ds(k, _CHUNK), :] = rows


def _tc_gather(emb_t, cb, rr_b):
    grid_spec = pltpu.PrefetchScalarGridSpec(
        num_scalar_prefetch=1,
        grid=(_GSTEPS,),
        in_specs=[
            pl.BlockSpec((1, _GBLK, EMBED), lambda s, cb: (s, 0, 0)),
            pl.BlockSpec(memory_space=pl.ANY),
        ],
        out_specs=pl.BlockSpec((_GBLK, EMBED), lambda s, cb: (s, 0)),
        scratch_shapes=[
            pltpu.VMEM((2, _GBLK, EMBED, 128), jnp.float32),
            pltpu.SemaphoreType.DMA((2,)),
        ],
    )
    return pl.pallas_call(
        _gather_body,
        grid_spec=grid_spec,
        out_shape=jax.ShapeDtypeStruct((NIDX, EMBED), jnp.float32),
    )(cb, rr_b, emb_t)


def _log_sigmoid(x):
    return jnp.minimum(x, 0.0) - jnp.log1p(jnp.exp(-jnp.abs(x)))


def _mm_body(cen_ref, ctx_ref, out_ref):
    prod = lax.dot_general(
        cen_ref[...], ctx_ref[...],
        (((1,), (1,)), ((), ())),
        preferred_element_type=jnp.float32,
    )
    out_ref[...] = _log_sigmoid(prod)


_BM = 512


def _tc_matmul(cen, ctx):
    return pl.pallas_call(
        _mm_body,
        grid=(B // _BM,),
        in_specs=[
            pl.BlockSpec((_BM, EMBED), lambda i: (i, 0)),
            pl.BlockSpec((B, EMBED), lambda i: (0, 0)),
        ],
        out_specs=pl.BlockSpec((_BM, B), lambda i: (i, 0)),
        out_shape=jax.ShapeDtypeStruct((B, B), jnp.float32),
    )(cen, ctx)


@jax.jit
def kernel(center_id, context_id, embeddings):
    ids = jnp.concatenate([center_id.astype(jnp.int32),
                           context_id.astype(jnp.int32)])
    cb = (ids // 128) * 128
    rr_b = jnp.broadcast_to(
        (ids % 128).reshape(_GSTEPS, _GBLK)[:, :, None],
        (_GSTEPS, _GBLK, EMBED))
    out = _tc_gather(embeddings.T, cb, rr_b)
    return _tc_matmul(out[:B], out[B:])


# speedup vs baseline: 2.3287x; 1.0200x over previous
import jax
import jax.numpy as jnp
from jax import lax
from jax.experimental import pallas as pl
from jax.experimental.pallas import tpu as pltpu

V = 1000000
EMBED = 32
B = 4096
NIDX = 2 * B

_GBLK = 128
_GSTEPS = NIDX // _GBLK
_CHUNK = 64
_BM = 512
_NSTRIPE = B // _BM
_S0 = _GSTEPS // 2 + 4
_GRID = _S0 + 4 * (_NSTRIPE - 1) + 1


def _log_sigmoid(x):
    return jnp.minimum(x, 0.0) - jnp.log1p(jnp.exp(-jnp.abs(x)))


def _body(cb_smem, rr_ref, emb_t_hbm, out_ref, buf, gath, sem):
    s = pl.program_id(0)

    def fire(step, slot):
        for k in range(_GBLK):
            pltpu.make_async_copy(
                emb_t_hbm.at[
                    :, pl.ds(pl.multiple_of(cb_smem[step * _GBLK + k], 128),
                             128)],
                buf.at[slot, k],
                sem.at[slot],
            ).start()

    @pl.when(s == 0)
    def _():
        fire(s, 0)

    @pl.when(s + 1 < _GSTEPS)
    def _():
        fire(s + 1, (s + 1) % 2)

    @pl.when(s < _GSTEPS)
    def _():
        slot = s % 2
        for k in range(_GBLK):
            pltpu.make_async_copy(
                emb_t_hbm.at[:, pl.ds(0, 128)],
                buf.at[slot, k],
                sem.at[slot],
            ).wait()
        for k in range(0, _GBLK, _CHUNK):
            chunk = buf[slot, pl.ds(k, _CHUNK)]
            idxc = rr_ref[0, pl.ds(k, _CHUNK), :][..., None]
            rows = jnp.take_along_axis(chunk, idxc, axis=2)[..., 0]
            gath[pl.ds(pl.multiple_of(s * _GBLK + k, 8), _CHUNK), :] = rows

    is_stripe = jnp.logical_and(s >= _S0, (s - _S0) % 4 == 0)

    @pl.when(is_stripe)
    def _():
        i = (s - _S0) // 4
        cen = gath[pl.ds(pl.multiple_of(B + i * _BM, 8), _BM), :]
        ctx = gath[pl.ds(0, B), :]
        prod = lax.dot_general(
            cen, ctx, (((1,), (1,)), ((), ())),
            preferred_element_type=jnp.float32,
        )
        out_ref[...] = _log_sigmoid(prod)


@jax.jit
def kernel(center_id, context_id, embeddings):
    ids = jnp.concatenate([context_id.astype(jnp.int32),
                           center_id.astype(jnp.int32)])
    cb = jnp.concatenate([(ids // 128) * 128,
                          jnp.zeros((_GRID + 1) * _GBLK - NIDX, jnp.int32)])
    rr_b = jnp.broadcast_to(
        (ids % 128).reshape(_GSTEPS, _GBLK)[:, :, None],
        (_GSTEPS, _GBLK, EMBED))

    def stripe_of(s):
        return jnp.clip((s - _S0) // 4, 0, _NSTRIPE - 1)

    grid_spec = pltpu.PrefetchScalarGridSpec(
        num_scalar_prefetch=1,
        grid=(_GRID,),
        in_specs=[
            pl.BlockSpec((1, _GBLK, EMBED), lambda s, cb: (s % _GSTEPS, 0, 0)),
            pl.BlockSpec(memory_space=pl.ANY),
        ],
        out_specs=pl.BlockSpec((_BM, B), lambda s, cb: (stripe_of(s), 0)),
        scratch_shapes=[
            pltpu.VMEM((2, _GBLK, EMBED, 128), jnp.float32),
            pltpu.VMEM((NIDX, EMBED), jnp.float32),
            pltpu.SemaphoreType.DMA((2,)),
        ],
    )
    return pl.pallas_call(
        _body,
        grid_spec=grid_spec,
        out_shape=jax.ShapeDtypeStruct((B, B), jnp.float32),
    )(cb, rr_b, embeddings.T)
